# baseline (speedup 1.0000x reference)
import jax
import jax.numpy as jnp
from jax.experimental import pallas as pl

MP_ROUNDS = 3
N_NODES = 10000


def kernel(node_features, edge_attr, edge_index, bi_edges_indx, W_enc, b_enc, W_msg, b_msg, W_upd, b_upd, W_dec, b_dec):
    src = edge_index[0]
    dst = edge_index[1]
    nodes = jax.nn.relu(node_features @ W_enc + b_enc)
    edges = jax.nn.relu(edge_attr @ W_enc + b_enc)
    for _ in range(MP_ROUNDS):
        m_in = jnp.concatenate([nodes[src], nodes[dst], edges], axis=1)
        msg = jax.nn.relu(m_in @ W_msg + b_msg)
        agg = jax.ops.segment_sum(msg, dst, num_segments=N_NODES)
        nodes = jax.nn.relu(jnp.concatenate([nodes, agg], axis=1) @ W_upd + b_upd)
        edges = msg
    i = bi_edges_indx[0]
    j = bi_edges_indx[1]
    avg = 0.5 * (edges[i] + edges[j])
    edges = edges.at[i].set(avg)
    edges = edges.at[j].set(avg)
    vals = (edges @ W_dec + b_dec)[:, 0]
    low_tri = jnp.where(src >= dst, vals, jnp.zeros_like(vals))
    return low_tri


# speedup vs baseline: 1.0720x; 1.0720x over previous
import functools

import jax
import jax.numpy as jnp
from jax import lax
from jax.experimental import pallas as pl
from jax.experimental.pallas import tpu as pltpu

MP_ROUNDS = 3
N = 10000
E = 320000
D = 128
H = 64

EB = 4000


def _enc_proj_body(x_ref, we_ref, be_ref, w3_ref, o_ref):
    h = jnp.maximum(
        jnp.dot(x_ref[...], we_ref[...], preferred_element_type=jnp.float32)
        + be_ref[...], 0.0)
    o_ref[...] = jnp.dot(h, w3_ref[...], preferred_element_type=jnp.float32)


def _edge_encode_project(ea, W_enc, b_enc, W3):
    return pl.pallas_call(
        _enc_proj_body,
        grid=(E // EB,),
        in_specs=[
            pl.BlockSpec((EB, D), lambda i: (i, 0)),
            pl.BlockSpec((D, H), lambda i: (0, 0)),
            pl.BlockSpec((1, H), lambda i: (0, 0)),
            pl.BlockSpec((H, H), lambda i: (0, 0)),
        ],
        out_specs=pl.BlockSpec((EB, H), lambda i: (i, 0)),
        out_shape=jax.ShapeDtypeStruct((E, H), jnp.float32),
    )(ea, W_enc, b_enc.reshape(1, H), W3)


def _proj_body(x_ref, w3_ref, o_ref):
    o_ref[...] = jnp.dot(x_ref[...], w3_ref[...],
                         preferred_element_type=jnp.float32)


def _edge_project(msg, W3):
    return pl.pallas_call(
        _proj_body,
        grid=(E // EB,),
        in_specs=[
            pl.BlockSpec((EB, H), lambda i: (i, 0)),
            pl.BlockSpec((H, H), lambda i: (0, 0)),
        ],
        out_specs=pl.BlockSpec((EB, H), lambda i: (i, 0)),
        out_shape=jax.ShapeDtypeStruct((E, H), jnp.float32),
    )(msg, W3)


def _node_init_body(nf_ref, we_ref, be_ref, w1_ref, w2_ref, bm_ref,
                    nodes_ref, ns1b_ref, nd2_ref):
    nodes = jnp.maximum(
        jnp.dot(nf_ref[...], we_ref[...], preferred_element_type=jnp.float32)
        + be_ref[...], 0.0)
    nodes_ref[...] = nodes
    ns1b_ref[...] = jnp.dot(nodes, w1_ref[...],
                            preferred_element_type=jnp.float32) + bm_ref[...]
    nd2_ref[...] = jnp.dot(nodes, w2_ref[...],
                           preferred_element_type=jnp.float32)


def _node_init(nf, W_enc, b_enc, W1, W2, b_msg):
    return pl.pallas_call(
        _node_init_body,
        out_shape=[jax.ShapeDtypeStruct((N, H), jnp.float32)] * 3,
    )(nf, W_enc, b_enc.reshape(1, H), W1, W2, b_msg.reshape(1, H))


def _node_upd_body(nodes_ref, agg_ref, wu1_ref, wu2_ref, bu_ref,
                   w1_ref, w2_ref, bm_ref, nn_ref, ns1b_ref, nd2_ref):
    nn = jnp.maximum(
        jnp.dot(nodes_ref[...], wu1_ref[...], preferred_element_type=jnp.float32)
        + jnp.dot(agg_ref[...], wu2_ref[...], preferred_element_type=jnp.float32)
        + bu_ref[...], 0.0)
    nn_ref[...] = nn
    ns1b_ref[...] = jnp.dot(nn, w1_ref[...],
                            preferred_element_type=jnp.float32) + bm_ref[...]
    nd2_ref[...] = jnp.dot(nn, w2_ref[...],
                           preferred_element_type=jnp.float32)


def _node_update(nodes, agg, Wu1, Wu2, b_upd, W1, W2, b_msg):
    return pl.pallas_call(
        _node_upd_body,
        out_shape=[jax.ShapeDtypeStruct((N, H), jnp.float32)] * 3,
    )(nodes, agg, Wu1, Wu2, b_upd.reshape(1, H), W1, W2, b_msg.reshape(1, H))


def kernel(node_features, edge_attr, edge_index, bi_edges_indx,
           W_enc, b_enc, W_msg, b_msg, W_upd, b_upd, W_dec, b_dec):
    src = edge_index[0]
    dst = edge_index[1]
    W1, W2, W3 = W_msg[:H], W_msg[H:2 * H], W_msg[2 * H:]
    Wu1, Wu2 = W_upd[:H], W_upd[H:]

    t = _edge_encode_project(edge_attr, W_enc, b_enc, W3)
    nodes, ns1b, nd2 = _node_init(node_features, W_enc, b_enc, W1, W2, b_msg)

    msg = None
    for r in range(MP_ROUNDS):
        msg = jax.nn.relu(jnp.take(ns1b, src, axis=0)
                          + jnp.take(nd2, dst, axis=0) + t)
        if r < MP_ROUNDS - 1:
            agg = jax.ops.segment_sum(msg, dst, num_segments=N)
            nodes, ns1b, nd2 = _node_update(nodes, agg, Wu1, Wu2, b_upd,
                                            W1, W2, b_msg)
            t = _edge_project(msg, W3)

    vals = (msg @ W_dec + b_dec)[:, 0]
    i = bi_edges_indx[0]
    j = bi_edges_indx[1]
    num_pairs = i.shape[0]
    targets = jnp.concatenate([i, j])
    ranks = jnp.arange(2 * num_pairs, dtype=jnp.int32)
    winner = jax.ops.segment_max(ranks, targets, num_segments=E)
    hit = winner >= 0
    q = jnp.where(hit, winner, 0) % num_pairs
    avg_vals = 0.5 * (vals[i] + vals[j])
    vals = jnp.where(hit, avg_vals[q], vals)
    return jnp.where(src >= dst, vals, jnp.zeros_like(vals))


# speedup vs baseline: 1.6948x; 1.5810x over previous
import functools

import jax
import jax.numpy as jnp
from jax import lax
from jax.experimental import pallas as pl
from jax.experimental.pallas import tpu as pltpu
from jax.experimental.pallas import tpu_sc as plsc

MP_ROUNDS = 3
N = 10000
E = 320000
D = 128
H = 64

EB = 4000

_NWORK = 32
_EPW = E // _NWORK
_W = 80
_NWIN = _EPW // _W
_ZR = 624


def _sc_round_body(do_agg, nproj, t, src, dst, *rest):
    if do_agg:
        msg_out, part_out, src_v, dst_v, gs, gd, tb, acc = rest
    else:
        msg_out, src_v, dst_v, gs, gd, tb = rest
        part_out = acc = None
    c = lax.axis_index("c")
    s = lax.axis_index("s")
    wid = s * 2 + c
    base0 = wid * _EPW

    def _zero_gs(r, carry):
        for ch in range(8):
            gs[r, pl.ds(ch * 16, 16)] = jnp.zeros((16,), jnp.float32)
        return carry

    if do_agg:
        lax.fori_loop(0, _W, _zero_gs, 0)
        r0 = s * _ZR
        for k in range(7):
            pltpu.sync_copy(gs, acc.at[pl.ds(r0 + k * _W, _W)])
        pltpu.sync_copy(gs.at[pl.ds(0, 64)], acc.at[pl.ds(r0 + 7 * _W, 64)])

        @pl.when(s == 15)
        def _zero_tail():
            pltpu.sync_copy(gs.at[pl.ds(0, 16)], acc.at[pl.ds(16 * _ZR, 16)])

        plsc.subcore_barrier()

    def _window(w, carry):
        base = base0 + w * _W
        pltpu.sync_copy(src.at[pl.ds(base, _W)], src_v)
        pltpu.sync_copy(dst.at[pl.ds(base, _W)], dst_v)
        pltpu.sync_copy(t.at[pl.ds(base, _W)], tb)
        pltpu.sync_copy(nproj.at[src_v], gs)
        pltpu.sync_copy(nproj.at[dst_v], gd)

        def _rbody(r, inner):
            for ch in range(4):
                sl = pl.ds(ch * 16, 16)
                sr = pl.ds(64 + ch * 16, 16)
                v = jnp.maximum(gs[r, sl] + gd[r, sr] + tb[r, sl], 0.0)
                tb[r, sl] = v
                if do_agg:
                    gs[r, sl] = v
            return inner

        lax.fori_loop(0, _W, _rbody, 0)
        pltpu.sync_copy(tb, msg_out.at[pl.ds(base, _W)])
        if do_agg:
            pltpu.sync_copy(gs, acc.at[dst_v], add=True)
        return carry

    lax.fori_loop(0, _NWIN, _window, 0)

    if do_agg:
        plsc.subcore_barrier()
        r0 = s * _ZR
        pltpu.sync_copy(acc.at[pl.ds(r0, _ZR)], part_out.at[c, pl.ds(r0, _ZR)])

        @pl.when(s == 15)
        def _unload_tail():
            pltpu.sync_copy(acc.at[pl.ds(16 * _ZR, 16)],
                            part_out.at[c, pl.ds(16 * _ZR, 16)])


def _sc_round(nproj, t, src, dst, do_agg):
    out_type = [jax.ShapeDtypeStruct((E, H), jnp.float32)]
    if do_agg:
        out_type.append(jax.ShapeDtypeStruct((2, N, 2 * H), jnp.float32))
    scratch = [
        pltpu.VMEM((_W,), jnp.int32),
        pltpu.VMEM((_W,), jnp.int32),
        pltpu.VMEM((_W, 2 * H), jnp.float32),
        pltpu.VMEM((_W, 2 * H), jnp.float32),
        pltpu.VMEM((_W, H), jnp.float32),
    ]
    if do_agg:
        scratch.append(pltpu.VMEM_SHARED((N, 2 * H), jnp.float32))
    mesh = plsc.VectorSubcoreMesh(core_axis_name="c", subcore_axis_name="s")
    fn = pl.kernel(
        functools.partial(_sc_round_body, do_agg),
        mesh=mesh,
        out_type=out_type,
        scratch_types=scratch,
    )
    return fn(nproj, t, src, dst)


def _enc_proj_body(x_ref, we_ref, be_ref, w3_ref, o_ref):
    h = jnp.maximum(
        jnp.dot(x_ref[...], we_ref[...], preferred_element_type=jnp.float32)
        + be_ref[...], 0.0)
    o_ref[...] = jnp.dot(h, w3_ref[...], preferred_element_type=jnp.float32)


def _edge_encode_project(ea, W_enc, b_enc, W3):
    return pl.pallas_call(
        _enc_proj_body,
        grid=(E // EB,),
        in_specs=[
            pl.BlockSpec((EB, D), lambda i: (i, 0)),
            pl.BlockSpec((D, H), lambda i: (0, 0)),
            pl.BlockSpec((1, H), lambda i: (0, 0)),
            pl.BlockSpec((H, H), lambda i: (0, 0)),
        ],
        out_specs=pl.BlockSpec((EB, H), lambda i: (i, 0)),
        out_shape=jax.ShapeDtypeStruct((E, H), jnp.float32),
    )(ea, W_enc, b_enc.reshape(1, H), W3)


def _proj_body(x_ref, w3_ref, o_ref):
    o_ref[...] = jnp.dot(x_ref[...], w3_ref[...],
                         preferred_element_type=jnp.float32)


def _edge_project(msg, W3):
    return pl.pallas_call(
        _proj_body,
        grid=(E // EB,),
        in_specs=[
            pl.BlockSpec((EB, H), lambda i: (i, 0)),
            pl.BlockSpec((H, H), lambda i: (0, 0)),
        ],
        out_specs=pl.BlockSpec((EB, H), lambda i: (i, 0)),
        out_shape=jax.ShapeDtypeStruct((E, H), jnp.float32),
    )(msg, W3)


def _node_init_body(nf_ref, we_ref, be_ref, w1_ref, w2_ref, bm_ref,
                    nodes_ref, nproj_ref):
    nodes = jnp.maximum(
        jnp.dot(nf_ref[...], we_ref[...], preferred_element_type=jnp.float32)
        + be_ref[...], 0.0)
    nodes_ref[...] = nodes
    a = jnp.dot(nodes, w1_ref[...], preferred_element_type=jnp.float32) \
        + bm_ref[...]
    b = jnp.dot(nodes, w2_ref[...], preferred_element_type=jnp.float32)
    nproj_ref[...] = jnp.concatenate([a, b], axis=1)


def _node_init(nf, W_enc, b_enc, W1, W2, b_msg):
    return pl.pallas_call(
        _node_init_body,
        out_shape=[jax.ShapeDtypeStruct((N, H), jnp.float32),
                   jax.ShapeDtypeStruct((N, 2 * H), jnp.float32)],
    )(nf, W_enc, b_enc.reshape(1, H), W1, W2, b_msg.reshape(1, H))


def _node_upd_body(nodes_ref, part_ref, wu1_ref, wu2_ref, bu_ref,
                   w1_ref, w2_ref, bm_ref, nn_ref, nproj_ref):
    agg = part_ref[0, :, :H] + part_ref[1, :, :H]
    nn = jnp.maximum(
        jnp.dot(nodes_ref[...], wu1_ref[...], preferred_element_type=jnp.float32)
        + jnp.dot(agg, wu2_ref[...], preferred_element_type=jnp.float32)
        + bu_ref[...], 0.0)
    nn_ref[...] = nn
    a = jnp.dot(nn, w1_ref[...], preferred_element_type=jnp.float32) \
        + bm_ref[...]
    b = jnp.dot(nn, w2_ref[...], preferred_element_type=jnp.float32)
    nproj_ref[...] = jnp.concatenate([a, b], axis=1)


def _node_update(nodes, partials, Wu1, Wu2, b_upd, W1, W2, b_msg):
    return pl.pallas_call(
        _node_upd_body,
        out_shape=[jax.ShapeDtypeStruct((N, H), jnp.float32),
                   jax.ShapeDtypeStruct((N, 2 * H), jnp.float32)],
    )(nodes, partials, Wu1, Wu2, b_upd.reshape(1, H), W1, W2,
      b_msg.reshape(1, H))


def kernel(node_features, edge_attr, edge_index, bi_edges_indx,
           W_enc, b_enc, W_msg, b_msg, W_upd, b_upd, W_dec, b_dec):
    src = edge_index[0]
    dst = edge_index[1]
    W1, W2, W3 = W_msg[:H], W_msg[H:2 * H], W_msg[2 * H:]
    Wu1, Wu2 = W_upd[:H], W_upd[H:]

    t = _edge_encode_project(edge_attr, W_enc, b_enc, W3)
    nodes, nproj = _node_init(node_features, W_enc, b_enc, W1, W2, b_msg)

    msg = None
    for r in range(MP_ROUNDS):
        if r < MP_ROUNDS - 1:
            msg, partials = _sc_round(nproj, t, src, dst, do_agg=True)
            nodes, nproj = _node_update(nodes, partials, Wu1, Wu2, b_upd,
                                        W1, W2, b_msg)
            t = _edge_project(msg, W3)
        else:
            (msg,) = _sc_round(nproj, t, src, dst, do_agg=False)

    vals = (msg @ W_dec + b_dec)[:, 0]
    i = bi_edges_indx[0]
    j = bi_edges_indx[1]
    num_pairs = i.shape[0]
    targets = jnp.concatenate([i, j])
    ranks = jnp.arange(2 * num_pairs, dtype=jnp.int32)
    winner = jax.ops.segment_max(ranks, targets, num_segments=E)
    hit = winner >= 0
    q = jnp.where(hit, winner, 0) % num_pairs
    avg_vals = 0.5 * (vals[i] + vals[j])
    vals = jnp.where(hit, avg_vals[q], vals)
    return jnp.where(src >= dst, vals, jnp.zeros_like(vals))


# speedup vs baseline: 2.4535x; 1.4476x over previous
import functools

import jax
import jax.numpy as jnp
from jax import lax
from jax.experimental import pallas as pl
from jax.experimental.pallas import tpu as pltpu
from jax.experimental.pallas import tpu_sc as plsc

MP_ROUNDS = 3
N = 10000
E = 320000
D = 128
H = 64

EB = 4000

_NWORK = 32
_EPW = E // _NWORK
_W = 80
_NWIN = _EPW // _W
_ZR = 624


def _sc_round_body(do_agg, nproj, t, src, dst, *rest):
    if do_agg:
        msg_out, part_out, src_v, dst_v, gs, gd, tb, acc = rest
    else:
        msg_out, src_v, dst_v, gs, gd, tb = rest
        part_out = acc = None
    c = lax.axis_index("c")
    s = lax.axis_index("s")
    wid = s * 2 + c
    base0 = wid * _EPW

    def _zero_gs(r, carry):
        for ch in range(8):
            gs[r, pl.ds(ch * 16, 16)] = jnp.zeros((16,), jnp.float32)
        return carry

    if do_agg:
        lax.fori_loop(0, _W, _zero_gs, 0)
        r0 = s * _ZR
        for k in range(7):
            pltpu.sync_copy(gs, acc.at[pl.ds(r0 + k * _W, _W)])
        pltpu.sync_copy(gs.at[pl.ds(0, 64)], acc.at[pl.ds(r0 + 7 * _W, 64)])

        @pl.when(s == 15)
        def _zero_tail():
            pltpu.sync_copy(gs.at[pl.ds(0, 16)], acc.at[pl.ds(16 * _ZR, 16)])

        plsc.subcore_barrier()

    def _window(w, carry):
        base = base0 + w * _W
        pltpu.sync_copy(src.at[pl.ds(base, _W)], src_v)
        pltpu.sync_copy(dst.at[pl.ds(base, _W)], dst_v)
        pltpu.sync_copy(t.at[pl.ds(base, _W)], tb)
        pltpu.sync_copy(nproj.at[src_v], gs)
        pltpu.sync_copy(nproj.at[dst_v], gd)

        def _rbody(r, inner):
            for ch in range(4):
                sl = pl.ds(ch * 16, 16)
                sr = pl.ds(64 + ch * 16, 16)
                v = jnp.maximum(gs[r, sl] + gd[r, sr] + tb[r, sl], 0.0)
                tb[r, sl] = v
                if do_agg:
                    gs[r, sl] = v
            return inner

        lax.fori_loop(0, _W, _rbody, 0)
        pltpu.sync_copy(tb, msg_out.at[pl.ds(base, _W)])
        if do_agg:
            pltpu.sync_copy(gs, acc.at[dst_v], add=True)
        return carry

    lax.fori_loop(0, _NWIN, _window, 0)

    if do_agg:
        plsc.subcore_barrier()
        r0 = s * _ZR
        pltpu.sync_copy(acc.at[pl.ds(r0, _ZR)], part_out.at[c, pl.ds(r0, _ZR)])

        @pl.when(s == 15)
        def _unload_tail():
            pltpu.sync_copy(acc.at[pl.ds(16 * _ZR, 16)],
                            part_out.at[c, pl.ds(16 * _ZR, 16)])


def _sc_round(nproj, t, src, dst, do_agg):
    out_type = [jax.ShapeDtypeStruct((E, H), jnp.float32)]
    if do_agg:
        out_type.append(jax.ShapeDtypeStruct((2, N, 2 * H), jnp.float32))
    scratch = [
        pltpu.VMEM((_W,), jnp.int32),
        pltpu.VMEM((_W,), jnp.int32),
        pltpu.VMEM((_W, 2 * H), jnp.float32),
        pltpu.VMEM((_W, 2 * H), jnp.float32),
        pltpu.VMEM((_W, H), jnp.float32),
    ]
    if do_agg:
        scratch.append(pltpu.VMEM_SHARED((N, 2 * H), jnp.float32))
    mesh = plsc.VectorSubcoreMesh(core_axis_name="c", subcore_axis_name="s")
    fn = pl.kernel(
        functools.partial(_sc_round_body, do_agg),
        mesh=mesh,
        out_type=out_type,
        scratch_types=scratch,
    )
    return fn(nproj, t, src, dst)


def _enc_proj_body(x_ref, we_ref, be_ref, w3_ref, o_ref):
    h = jnp.maximum(
        jnp.dot(x_ref[...], we_ref[...], preferred_element_type=jnp.float32)
        + be_ref[...], 0.0)
    o_ref[...] = jnp.dot(h, w3_ref[...], preferred_element_type=jnp.float32)


def _edge_encode_project(ea, W_enc, b_enc, W3):
    return pl.pallas_call(
        _enc_proj_body,
        grid=(E // EB,),
        in_specs=[
            pl.BlockSpec((EB, D), lambda i: (i, 0)),
            pl.BlockSpec((D, H), lambda i: (0, 0)),
            pl.BlockSpec((1, H), lambda i: (0, 0)),
            pl.BlockSpec((H, H), lambda i: (0, 0)),
        ],
        out_specs=pl.BlockSpec((EB, H), lambda i: (i, 0)),
        out_shape=jax.ShapeDtypeStruct((E, H), jnp.float32),
    )(ea, W_enc, b_enc.reshape(1, H), W3)


def _proj_body(x_ref, w3_ref, o_ref):
    o_ref[...] = jnp.dot(x_ref[...], w3_ref[...],
                         preferred_element_type=jnp.float32)


def _edge_project(msg, W3):
    return pl.pallas_call(
        _proj_body,
        grid=(E // EB,),
        in_specs=[
            pl.BlockSpec((EB, H), lambda i: (i, 0)),
            pl.BlockSpec((H, H), lambda i: (0, 0)),
        ],
        out_specs=pl.BlockSpec((EB, H), lambda i: (i, 0)),
        out_shape=jax.ShapeDtypeStruct((E, H), jnp.float32),
    )(msg, W3)


def _node_init_body(nf_ref, we_ref, be_ref, w1_ref, w2_ref, bm_ref,
                    nodes_ref, nproj_ref):
    nodes = jnp.maximum(
        jnp.dot(nf_ref[...], we_ref[...], preferred_element_type=jnp.float32)
        + be_ref[...], 0.0)
    nodes_ref[...] = nodes
    a = jnp.dot(nodes, w1_ref[...], preferred_element_type=jnp.float32) \
        + bm_ref[...]
    b = jnp.dot(nodes, w2_ref[...], preferred_element_type=jnp.float32)
    nproj_ref[...] = jnp.concatenate([a, b], axis=1)


def _node_init(nf, W_enc, b_enc, W1, W2, b_msg):
    return pl.pallas_call(
        _node_init_body,
        out_shape=[jax.ShapeDtypeStruct((N, H), jnp.float32),
                   jax.ShapeDtypeStruct((N, 2 * H), jnp.float32)],
    )(nf, W_enc, b_enc.reshape(1, H), W1, W2, b_msg.reshape(1, H))


def _node_upd_body(nodes_ref, part_ref, wu1_ref, wu2_ref, bu_ref,
                   w1_ref, w2_ref, bm_ref, nn_ref, nproj_ref):
    agg = part_ref[0, :, :H] + part_ref[1, :, :H]
    nn = jnp.maximum(
        jnp.dot(nodes_ref[...], wu1_ref[...], preferred_element_type=jnp.float32)
        + jnp.dot(agg, wu2_ref[...], preferred_element_type=jnp.float32)
        + bu_ref[...], 0.0)
    nn_ref[...] = nn
    a = jnp.dot(nn, w1_ref[...], preferred_element_type=jnp.float32) \
        + bm_ref[...]
    b = jnp.dot(nn, w2_ref[...], preferred_element_type=jnp.float32)
    nproj_ref[...] = jnp.concatenate([a, b], axis=1)


def _node_update(nodes, partials, Wu1, Wu2, b_upd, W1, W2, b_msg):
    return pl.pallas_call(
        _node_upd_body,
        out_shape=[jax.ShapeDtypeStruct((N, H), jnp.float32),
                   jax.ShapeDtypeStruct((N, 2 * H), jnp.float32)],
    )(nodes, partials, Wu1, Wu2, b_upd.reshape(1, H), W1, W2,
      b_msg.reshape(1, H))


_PP = 160000
_PPW = _PP // _NWORK
_WP = 128
_NWF = _PPW // _WP
_TAIL = _PPW - _NWF * _WP
_RT = E // _NWORK


def _sc_pair_avg_body(vals, iarr, jarr, a_out, i_v, j_v, vi, vj, sem):
    c = lax.axis_index("c")
    s = lax.axis_index("s")
    wid = s * 2 + c
    base0 = wid * _PPW

    def _win(w, carry):
        base = base0 + w * _WP
        pltpu.sync_copy(iarr.at[pl.ds(base, _WP)], i_v)
        pltpu.sync_copy(jarr.at[pl.ds(base, _WP)], j_v)
        g1 = pltpu.async_copy(vals.at[i_v], vi, sem)
        g2 = pltpu.async_copy(vals.at[j_v], vj, sem)
        g1.wait()
        g2.wait()
        for k in range(_WP // 16):
            sl = pl.ds(k * 16, 16)
            vi[sl] = 0.5 * (vi[sl] + vj[sl])
        pltpu.sync_copy(vi, a_out.at[pl.ds(base, _WP)])
        return carry

    lax.fori_loop(0, _NWF, _win, 0)

    base = base0 + _NWF * _WP
    pltpu.sync_copy(iarr.at[pl.ds(base, _TAIL)], i_v.at[pl.ds(0, _TAIL)])
    pltpu.sync_copy(jarr.at[pl.ds(base, _TAIL)], j_v.at[pl.ds(0, _TAIL)])
    pltpu.sync_copy(vals.at[i_v.at[pl.ds(0, _TAIL)]], vi.at[pl.ds(0, _TAIL)])
    pltpu.sync_copy(vals.at[j_v.at[pl.ds(0, _TAIL)]], vj.at[pl.ds(0, _TAIL)])
    vi[pl.ds(0, 16)] = 0.5 * (vi[pl.ds(0, 16)] + vj[pl.ds(0, 16)])
    pltpu.sync_copy(vi.at[pl.ds(0, _TAIL)], a_out.at[pl.ds(base, _TAIL)])


def _sc_pair_avg(vals, iarr, jarr):
    mesh = plsc.VectorSubcoreMesh(core_axis_name="c", subcore_axis_name="s")
    fn = pl.kernel(
        _sc_pair_avg_body,
        mesh=mesh,
        out_type=jax.ShapeDtypeStruct((_PP,), jnp.float32),
        scratch_types=[
            pltpu.VMEM((_WP,), jnp.int32),
            pltpu.VMEM((_WP,), jnp.int32),
            pltpu.VMEM((_WP,), jnp.float32),
            pltpu.VMEM((_WP,), jnp.float32),
            pltpu.SemaphoreType.DMA,
        ],
    )
    return fn(vals, iarr, jarr)


def _sc_pair_apply_body(vals, ind, wmax, aarr, out, v_l, indl, w_l, q_v, g_v,
                        sem):
    c = lax.axis_index("c")
    s = lax.axis_index("s")
    wid = s * 2 + c
    e0 = wid * _RT
    pltpu.sync_copy(vals.at[pl.ds(e0, _RT)], v_l)
    pltpu.sync_copy(ind.at[pl.ds(e0, _RT)], indl)
    pltpu.sync_copy(wmax.at[pl.ds(e0, _RT)], w_l)

    def _mkidx(k, carry):
        sl = pl.ds(k * 16, 16)
        r = w_l[sl].astype(jnp.int32)
        q = jnp.where(r >= _PP, r - _PP, r)
        q_v[sl] = jnp.where(r < 0, 0, q)
        return carry

    lax.fori_loop(0, _RT // 16, _mkidx, 0)
    handles = []
    for k in range(_RT // 128):
        sl = pl.ds(k * 128, 128)
        handles.append(pltpu.async_copy(aarr.at[q_v.at[sl]], g_v.at[sl], sem))
    tl = pl.ds((_RT // 128) * 128, _RT % 128)
    if _RT % 128:
        handles.append(pltpu.async_copy(aarr.at[q_v.at[tl]], g_v.at[tl], sem))
    for h in handles:
        h.wait()

    def _combine(k, carry):
        sl = pl.ds(k * 16, 16)
        hit = w_l[sl] >= 0.0
        res = jnp.where(hit, g_v[sl], v_l[sl])
        v_l[sl] = jnp.where(indl[sl] > 0.5, res, 0.0)
        return carry

    lax.fori_loop(0, _RT // 16, _combine, 0)
    pltpu.sync_copy(v_l, out.at[pl.ds(e0, _RT)])


def _sc_pair_apply(vals, ind, wmax, aarr):
    mesh = plsc.VectorSubcoreMesh(core_axis_name="c", subcore_axis_name="s")
    fn = pl.kernel(
        _sc_pair_apply_body,
        mesh=mesh,
        out_type=jax.ShapeDtypeStruct((E,), jnp.float32),
        scratch_types=[
            pltpu.VMEM((_RT,), jnp.float32),
            pltpu.VMEM((_RT,), jnp.float32),
            pltpu.VMEM((_RT,), jnp.float32),
            pltpu.VMEM((_RT,), jnp.int32),
            pltpu.VMEM((_RT,), jnp.float32),
            pltpu.SemaphoreType.DMA,
        ],
    )
    return fn(vals, ind, wmax, aarr)


def kernel(node_features, edge_attr, edge_index, bi_edges_indx,
           W_enc, b_enc, W_msg, b_msg, W_upd, b_upd, W_dec, b_dec):
    src = edge_index[0]
    dst = edge_index[1]
    W1, W2, W3 = W_msg[:H], W_msg[H:2 * H], W_msg[2 * H:]
    Wu1, Wu2 = W_upd[:H], W_upd[H:]

    t = _edge_encode_project(edge_attr, W_enc, b_enc, W3)
    nodes, nproj = _node_init(node_features, W_enc, b_enc, W1, W2, b_msg)

    msg = None
    for r in range(MP_ROUNDS):
        if r < MP_ROUNDS - 1:
            msg, partials = _sc_round(nproj, t, src, dst, do_agg=True)
            nodes, nproj = _node_update(nodes, partials, Wu1, Wu2, b_upd,
                                        W1, W2, b_msg)
            t = _edge_project(msg, W3)
        else:
            (msg,) = _sc_round(nproj, t, src, dst, do_agg=False)

    vals = (msg @ W_dec + b_dec)[:, 0]
    ind = (src >= dst).astype(jnp.float32)
    i = bi_edges_indx[0]
    j = bi_edges_indx[1]
    avg = _sc_pair_avg(vals, i, j)
    targets = jnp.concatenate([i, j])
    ranks = jnp.arange(2 * _PP, dtype=jnp.float32)
    wmax = jnp.full((E,), -1.0, jnp.float32).at[targets].max(ranks)
    return _sc_pair_apply(vals, ind, wmax, avg)
